# baseline (speedup 1.0000x reference)
import jax
import jax.numpy as jnp
from jax import lax
from jax.experimental import pallas as pl
from jax.experimental.pallas import tpu as pltpu
from jax.experimental.pallas import tpu_sc as plsc

D = 128
LANES = 16
VPR = D // LANES
NCORES = 2
NSUB = 16
NW = NCORES * NSUB
CHUNK = 256
NBUF = 3
MAGIC = 8388608.0
INV7 = 1.0 / 7.0
TINY = 1e-30


def _compute_chunk(buf):
    @plsc.parallel_loop(0, CHUNK, unroll=2)
    def _(r):
        vs = [buf[r, pl.ds(j * LANES, LANES)] for j in range(VPR)]
        mn = jnp.minimum(jnp.minimum(jnp.minimum(vs[0], vs[1]),
                                     jnp.minimum(vs[2], vs[3])),
                         jnp.minimum(jnp.minimum(vs[4], vs[5]),
                                     jnp.minimum(vs[6], vs[7])))
        mx = jnp.maximum(jnp.maximum(jnp.maximum(vs[0], vs[1]),
                                     jnp.maximum(vs[2], vs[3])),
                         jnp.maximum(jnp.maximum(vs[4], vs[5]),
                                     jnp.maximum(vs[6], vs[7])))
        mn = jnp.full((LANES,), jnp.min(mn), jnp.float32)
        mx = jnp.full((LANES,), jnp.max(mx), jnp.float32)
        safe = jnp.maximum(mx - mn, TINY)
        inv = 7.0 / safe
        scale = safe * INV7
        for j in range(VPR):
            c = (vs[j] - mn) * inv
            rnd = (c + MAGIC) - MAGIC
            buf[r, pl.ds(j * LANES, LANES)] = mn + rnd * scale


def _sc_body(x_hbm, o_hbm, b0, b1, b2, si0, si1, si2, so0, so1, so2):
    rows_per_w = x_hbm.shape[0] // NW
    wid = lax.axis_index("s") * NCORES + lax.axis_index("c")
    base = wid * rows_per_w
    nchunks = rows_per_w // CHUNK
    bufs = [b0, b1, b2]
    sis = [si0, si1, si2]
    sos = [so0, so1, so2]

    def start_in(g, b):
        pltpu.async_copy(x_hbm.at[pl.ds(base + g * CHUNK, CHUNK)],
                         bufs[b], sis[b])

    def wait_in(b):
        pltpu.make_async_copy(x_hbm.at[pl.ds(base, CHUNK)],
                              bufs[b], sis[b]).wait()

    def start_out(g, b):
        pltpu.async_copy(bufs[b],
                         o_hbm.at[pl.ds(base + g * CHUNK, CHUNK)], sos[b])

    def wait_out(b):
        pltpu.make_async_copy(bufs[b],
                              o_hbm.at[pl.ds(base, CHUNK)], sos[b]).wait()

    start_in(0, 0)
    start_in(1, 1)
    for g in range(nchunks):
        b = g % NBUF
        wait_in(b)
        _compute_chunk(bufs[b])
        start_out(g, b)
        nxt = g + 2
        if nxt < nchunks:
            nb = nxt % NBUF
            if nxt >= NBUF:
                wait_out(nb)
            start_in(nxt, nb)
    for g in range(nchunks - NBUF, nchunks):
        wait_out(g % NBUF)


def _quantize_recon(x):
    n = x.shape[0]
    mesh = plsc.VectorSubcoreMesh(
        core_axis_name="c", subcore_axis_name="s",
        num_cores=NCORES, num_subcores=NSUB)
    return pl.kernel(
        _sc_body,
        out_type=jax.ShapeDtypeStruct((n, D), jnp.float32),
        mesh=mesh,
        scratch_types=[
            pltpu.VMEM((CHUNK, D), jnp.float32),
            pltpu.VMEM((CHUNK, D), jnp.float32),
            pltpu.VMEM((CHUNK, D), jnp.float32),
            pltpu.SemaphoreType.DMA,
            pltpu.SemaphoreType.DMA,
            pltpu.SemaphoreType.DMA,
            pltpu.SemaphoreType.DMA,
            pltpu.SemaphoreType.DMA,
            pltpu.SemaphoreType.DMA,
        ],
        compiler_params=pltpu.CompilerParams(needs_layout_passes=False),
    )(x)


def kernel(kv_states):
    batch, num_heads, seq_len, head_dim = kv_states.shape
    x = kv_states.astype(jnp.float32).reshape(-1, head_dim)
    recon = _quantize_recon(x)
    return recon.reshape(batch, num_heads, seq_len, head_dim)
